# baseline (speedup 1.0000x reference)
import math

import jax
import jax.numpy as jnp
import numpy as np
from jax.experimental import pallas as pl
from jax.experimental.pallas import tpu as pltpu


M = 4
ND = 708
NP = 1512
DIN = 512
DH = 256
DMID = (DIN + DH) // 2

BN = 256
DBLK = (ND + BN - 1) // BN
PBLK = (NP + BN - 1) // BN
NROWS = (DBLK + PBLK) * BN

F32 = jnp.float32
BF16 = jnp.bfloat16

SCALE = 1.0 / math.sqrt(float(DH))

_BLK = np.repeat(np.eye(M * M, dtype=np.float32), DH, axis=0)
_SEGREP = np.kron(np.eye(M, dtype=np.float32), np.ones((M, M), np.float32))
_SEL = np.tile(np.eye(M, dtype=np.float32), (M, 1))


def _branch_body(x_ref, att_ref, blk_ref, segrep_ref, sel_ref,
                 w1_ref, w2_ref, w3_ref, A_ref,
                 vrow_ref, vw_ref, b1_ref, b2_ref, b3_ref, vb_ref, beta_ref,
                 o_ref):
    z16s = []
    g16s = []
    zsum = None
    for m in range(M):
        xm = x_ref[m].astype(BF16) * att_ref[m].astype(BF16)
        h = jnp.dot(xm, w1_ref[m], preferred_element_type=F32)
        h = jnp.maximum(h.astype(BF16) + b1_ref[m].astype(BF16), 0.0)
        h = jnp.dot(h, w2_ref[m], preferred_element_type=F32)
        h = jnp.maximum(h.astype(BF16) + b2_ref[m].astype(BF16), 0.0)
        z = jnp.dot(h, w3_ref[m], preferred_element_type=F32) + b3_ref[m]
        zsum = z if zsum is None else zsum + z
        z16 = z.astype(BF16)
        z16s.append(z16)
        g = jnp.dot(z16, A_ref[:], preferred_element_type=F32) + vrow_ref[:]
        g16s.append(g.astype(BF16))

    p = jnp.concatenate(
        [g16s[i] * z16s[j] for i in range(M) for j in range(M)], axis=1)
    s = jnp.dot(p, blk_ref[:], preferred_element_type=F32)
    mx = jnp.max(s, axis=1, keepdims=True)
    e = jnp.exp(s - mx)
    d = jnp.dot(e, segrep_ref[:], preferred_element_type=F32)
    attn = e * (1.0 / d)
    wsum = jnp.dot(attn, sel_ref[:], preferred_element_type=F32)

    vin = None
    for j in range(M):
        c = wsum[:, j:j + 1].astype(BF16) * z16s[j]
        vin = c if vin is None else vin + c

    beta = beta_ref[0, 0]
    vout = jnp.dot(vin, vw_ref[:], preferred_element_type=F32) * (1.0 / M)
    o_ref[:] = zsum * (1.0 / M) + beta * (vout + vb_ref[:])


def _kernel_body(xd_ref, attd_ref, xp_ref, attp_ref, blk_ref, segrep_ref,
                 sel_ref,
                 dw1_ref, dw2_ref, dw3_ref, dqw_ref, dkwt_ref, dvw_ref,
                 db1_ref, db2_ref, db3_ref, dqb_ref, dvb_ref,
                 pw1_ref, pw2_ref, pw3_ref, pqw_ref, pkwt_ref, pvw_ref,
                 pb1_ref, pb2_ref, pb3_ref, pqb_ref, pvb_ref,
                 dbeta_ref, pbeta_ref,
                 o_ref,
                 dw1s, dw2s, dw3s, dAs, dvrows, dvws,
                 pw1s, pw2s, pw3s, pAs, pvrows, pvws):
    i = pl.program_id(0)

    @pl.when(i == 0)
    def _():
        dw1s[:] = dw1_ref[:].astype(BF16)
        dw2s[:] = dw2_ref[:].astype(BF16)
        dw3s[:] = dw3_ref[:].astype(BF16)
        dkwt16 = dkwt_ref[:].astype(BF16)
        dAs[:] = (jnp.dot(dqw_ref[:].astype(BF16), dkwt16,
                          preferred_element_type=F32) * SCALE).astype(BF16)
        dvrows[:] = jnp.dot(dqb_ref[:].astype(BF16), dkwt16,
                            preferred_element_type=F32) * SCALE
        dvws[:] = dvw_ref[:].astype(BF16)
        pw1s[:] = pw1_ref[:].astype(BF16)
        pw2s[:] = pw2_ref[:].astype(BF16)
        pw3s[:] = pw3_ref[:].astype(BF16)
        pkwt16 = pkwt_ref[:].astype(BF16)
        pAs[:] = (jnp.dot(pqw_ref[:].astype(BF16), pkwt16,
                          preferred_element_type=F32) * SCALE).astype(BF16)
        pvrows[:] = jnp.dot(pqb_ref[:].astype(BF16), pkwt16,
                            preferred_element_type=F32) * SCALE
        pvws[:] = pvw_ref[:].astype(BF16)

    @pl.when(i < DBLK)
    def _():
        _branch_body(xd_ref, attd_ref, blk_ref, segrep_ref, sel_ref,
                     dw1s, dw2s, dw3s, dAs,
                     dvrows, dvws, db1_ref, db2_ref, db3_ref, dvb_ref,
                     dbeta_ref, o_ref)

    @pl.when(i >= DBLK)
    def _():
        _branch_body(xp_ref, attp_ref, blk_ref, segrep_ref, sel_ref,
                     pw1s, pw2s, pw3s, pAs,
                     pvrows, pvws, pb1_ref, pb2_ref, pb3_ref, pvb_ref,
                     pbeta_ref, o_ref)


_VMEM = pl.BlockSpec(memory_space=pltpu.VMEM)
_SMEM = pl.BlockSpec(memory_space=pltpu.SMEM)


@jax.jit
def kernel(drug_fea_tensor, protein_fea_tensor, drug_att, protein_att, dW1,
           db1, dW2, db2, dW3, db3, pW1, pb1, pW2, pb2, pW3, pb3, QdW, Qdb,
           KdW, Kdb, VdW, Vdb, QpW, Qpb, KpW, Kpb, VpW, Vpb, beta_drug,
           beta_protein):
    out = pl.pallas_call(
        _kernel_body,
        grid=(DBLK + PBLK,),
        in_specs=(
            [pl.BlockSpec((M, BN, DIN), lambda i: (0, jnp.minimum(i, DBLK - 1), 0)),
             pl.BlockSpec((M, BN, 1), lambda i: (0, jnp.minimum(i, DBLK - 1), 0)),
             pl.BlockSpec((M, BN, DIN), lambda i: (0, jnp.maximum(i - DBLK, 0), 0)),
             pl.BlockSpec((M, BN, 1), lambda i: (0, jnp.maximum(i - DBLK, 0), 0))]
            + [_VMEM] * 25 + [_SMEM] * 2
        ),
        out_specs=pl.BlockSpec((BN, DH), lambda i: (i, 0)),
        out_shape=jax.ShapeDtypeStruct((NROWS, DH), F32),
        scratch_shapes=[
            pltpu.VMEM((M, DIN, DMID), BF16), pltpu.VMEM((M, DMID, DMID), BF16),
            pltpu.VMEM((M, DMID, DH), BF16), pltpu.VMEM((DH, DH), BF16),
            pltpu.VMEM((1, DH), F32), pltpu.VMEM((DH, DH), BF16),
            pltpu.VMEM((M, DIN, DMID), BF16), pltpu.VMEM((M, DMID, DMID), BF16),
            pltpu.VMEM((M, DMID, DH), BF16), pltpu.VMEM((DH, DH), BF16),
            pltpu.VMEM((1, DH), F32), pltpu.VMEM((DH, DH), BF16),
        ],
    )(drug_fea_tensor, drug_att, protein_fea_tensor, protein_att,
      jnp.asarray(_BLK, BF16), jnp.asarray(_SEGREP), jnp.asarray(_SEL),
      dW1, dW2, dW3, QdW, KdW.T, VdW,
      db1, db2, db3, jnp.reshape(Qdb, (1, DH)), jnp.reshape(Vdb, (1, DH)),
      pW1, pW2, pW3, QpW, KpW.T, VpW,
      pb1, pb2, pb3, jnp.reshape(Qpb, (1, DH)), jnp.reshape(Vpb, (1, DH)),
      jnp.reshape(beta_drug, (1, 1)), jnp.reshape(beta_protein, (1, 1)))

    drug_emb = out[:ND]
    protein_emb = out[DBLK * BN:DBLK * BN + NP]
    return (drug_emb, protein_emb)


# speedup vs baseline: 1.0145x; 1.0145x over previous
import math

import jax
import jax.numpy as jnp
import numpy as np
from jax.experimental import pallas as pl
from jax.experimental.pallas import tpu as pltpu


M = 4
ND = 708
NP = 1512
DIN = 512
DH = 256
DMID = (DIN + DH) // 2

BN = 384
DBLK = (ND + BN - 1) // BN
PBLK = (NP + BN - 1) // BN
NROWS = (DBLK + PBLK) * BN

F32 = jnp.float32
BF16 = jnp.bfloat16

SCALE = 1.0 / math.sqrt(float(DH))

_BLK = np.repeat(np.eye(M * M, dtype=np.float32), DH, axis=0)
_SEGREP = np.kron(np.eye(M, dtype=np.float32), np.ones((M, M), np.float32))
_SEL = np.tile(np.eye(M, dtype=np.float32), (M, 1))


def _branch_body(x_ref, att_ref, blk_ref, segrep_ref, sel_ref,
                 w1_ref, w2_ref, w3_ref, A_ref,
                 vrow_ref, vw_ref, b1_ref, b2_ref, b3_ref, vb_ref, beta_ref,
                 o_ref):
    z16s = []
    g16s = []
    zsum = None
    for m in range(M):
        xm = x_ref[m].astype(BF16) * att_ref[m].astype(BF16)
        h = jnp.dot(xm, w1_ref[m], preferred_element_type=F32)
        h = jnp.maximum(h.astype(BF16) + b1_ref[m].astype(BF16), 0.0)
        h = jnp.dot(h, w2_ref[m], preferred_element_type=F32)
        h = jnp.maximum(h.astype(BF16) + b2_ref[m].astype(BF16), 0.0)
        z = jnp.dot(h, w3_ref[m], preferred_element_type=F32) + b3_ref[m]
        zsum = z if zsum is None else zsum + z
        z16 = z.astype(BF16)
        z16s.append(z16)
        g = jnp.dot(z16, A_ref[:], preferred_element_type=F32) + vrow_ref[:]
        g16s.append(g.astype(BF16))

    p = jnp.concatenate(
        [g16s[i] * z16s[j] for i in range(M) for j in range(M)], axis=1)
    s = jnp.dot(p, blk_ref[:], preferred_element_type=F32)
    mx = jnp.max(s, axis=1, keepdims=True)
    e = jnp.exp(s - mx)
    d = jnp.dot(e, segrep_ref[:], preferred_element_type=F32)
    attn = e * (1.0 / d)
    wsum = jnp.dot(attn, sel_ref[:], preferred_element_type=F32)

    vin = None
    for j in range(M):
        c = wsum[:, j:j + 1].astype(BF16) * z16s[j]
        vin = c if vin is None else vin + c

    beta = beta_ref[0, 0]
    vout = jnp.dot(vin, vw_ref[:], preferred_element_type=F32) * (1.0 / M)
    o_ref[:] = zsum * (1.0 / M) + beta * (vout + vb_ref[:])


def _kernel_body(xd_ref, attd_ref, xp_ref, attp_ref, blk_ref, segrep_ref,
                 sel_ref,
                 dw1_ref, dw2_ref, dw3_ref, dqw_ref, dkwt_ref, dvw_ref,
                 db1_ref, db2_ref, db3_ref, dqb_ref, dvb_ref,
                 pw1_ref, pw2_ref, pw3_ref, pqw_ref, pkwt_ref, pvw_ref,
                 pb1_ref, pb2_ref, pb3_ref, pqb_ref, pvb_ref,
                 dbeta_ref, pbeta_ref,
                 o_ref,
                 dw1s, dw2s, dw3s, dAs, dvrows, dvws,
                 pw1s, pw2s, pw3s, pAs, pvrows, pvws):
    i = pl.program_id(0)

    @pl.when(i == 0)
    def _():
        dw1s[:] = dw1_ref[:].astype(BF16)
        dw2s[:] = dw2_ref[:].astype(BF16)
        dw3s[:] = dw3_ref[:].astype(BF16)
        dkwt16 = dkwt_ref[:].astype(BF16)
        dAs[:] = (jnp.dot(dqw_ref[:].astype(BF16), dkwt16,
                          preferred_element_type=F32) * SCALE).astype(BF16)
        dvrows[:] = jnp.dot(dqb_ref[:].astype(BF16), dkwt16,
                            preferred_element_type=F32) * SCALE
        dvws[:] = dvw_ref[:].astype(BF16)
        pw1s[:] = pw1_ref[:].astype(BF16)
        pw2s[:] = pw2_ref[:].astype(BF16)
        pw3s[:] = pw3_ref[:].astype(BF16)
        pkwt16 = pkwt_ref[:].astype(BF16)
        pAs[:] = (jnp.dot(pqw_ref[:].astype(BF16), pkwt16,
                          preferred_element_type=F32) * SCALE).astype(BF16)
        pvrows[:] = jnp.dot(pqb_ref[:].astype(BF16), pkwt16,
                            preferred_element_type=F32) * SCALE
        pvws[:] = pvw_ref[:].astype(BF16)

    @pl.when(i < DBLK)
    def _():
        _branch_body(xd_ref, attd_ref, blk_ref, segrep_ref, sel_ref,
                     dw1s, dw2s, dw3s, dAs,
                     dvrows, dvws, db1_ref, db2_ref, db3_ref, dvb_ref,
                     dbeta_ref, o_ref)

    @pl.when(i >= DBLK)
    def _():
        _branch_body(xp_ref, attp_ref, blk_ref, segrep_ref, sel_ref,
                     pw1s, pw2s, pw3s, pAs,
                     pvrows, pvws, pb1_ref, pb2_ref, pb3_ref, pvb_ref,
                     pbeta_ref, o_ref)


_VMEM = pl.BlockSpec(memory_space=pltpu.VMEM)
_SMEM = pl.BlockSpec(memory_space=pltpu.SMEM)


@jax.jit
def kernel(drug_fea_tensor, protein_fea_tensor, drug_att, protein_att, dW1,
           db1, dW2, db2, dW3, db3, pW1, pb1, pW2, pb2, pW3, pb3, QdW, Qdb,
           KdW, Kdb, VdW, Vdb, QpW, Qpb, KpW, Kpb, VpW, Vpb, beta_drug,
           beta_protein):
    out = pl.pallas_call(
        _kernel_body,
        grid=(DBLK + PBLK,),
        in_specs=(
            [pl.BlockSpec((M, BN, DIN), lambda i: (0, jnp.minimum(i, DBLK - 1), 0)),
             pl.BlockSpec((M, BN, 1), lambda i: (0, jnp.minimum(i, DBLK - 1), 0)),
             pl.BlockSpec((M, BN, DIN), lambda i: (0, jnp.maximum(i - DBLK, 0), 0)),
             pl.BlockSpec((M, BN, 1), lambda i: (0, jnp.maximum(i - DBLK, 0), 0))]
            + [_VMEM] * 25 + [_SMEM] * 2
        ),
        out_specs=pl.BlockSpec((BN, DH), lambda i: (i, 0)),
        out_shape=jax.ShapeDtypeStruct((NROWS, DH), F32),
        scratch_shapes=[
            pltpu.VMEM((M, DIN, DMID), BF16), pltpu.VMEM((M, DMID, DMID), BF16),
            pltpu.VMEM((M, DMID, DH), BF16), pltpu.VMEM((DH, DH), BF16),
            pltpu.VMEM((1, DH), F32), pltpu.VMEM((DH, DH), BF16),
            pltpu.VMEM((M, DIN, DMID), BF16), pltpu.VMEM((M, DMID, DMID), BF16),
            pltpu.VMEM((M, DMID, DH), BF16), pltpu.VMEM((DH, DH), BF16),
            pltpu.VMEM((1, DH), F32), pltpu.VMEM((DH, DH), BF16),
        ],
    )(drug_fea_tensor, drug_att, protein_fea_tensor, protein_att,
      jnp.asarray(_BLK, BF16), jnp.asarray(_SEGREP), jnp.asarray(_SEL),
      dW1, dW2, dW3, QdW, KdW.T, VdW,
      db1, db2, db3, jnp.reshape(Qdb, (1, DH)), jnp.reshape(Vdb, (1, DH)),
      pW1, pW2, pW3, QpW, KpW.T, VpW,
      pb1, pb2, pb3, jnp.reshape(Qpb, (1, DH)), jnp.reshape(Vpb, (1, DH)),
      jnp.reshape(beta_drug, (1, 1)), jnp.reshape(beta_protein, (1, 1)))

    drug_emb = out[:ND]
    protein_emb = out[DBLK * BN:DBLK * BN + NP]
    return (drug_emb, protein_emb)


# speedup vs baseline: 1.1131x; 1.0972x over previous
import math

import jax
import jax.numpy as jnp
import numpy as np
from jax.experimental import pallas as pl
from jax.experimental.pallas import tpu as pltpu


M = 4
ND = 708
NP = 1512
DIN = 512
DH = 256
DMID = (DIN + DH) // 2

BN = 384
DBLK = (ND + BN - 1) // BN
PBLK = (NP + BN - 1) // BN
NROWS = (DBLK + PBLK) * BN

F32 = jnp.float32
BF16 = jnp.bfloat16

SCALE = 1.0 / math.sqrt(float(DH))

_BLK = np.repeat(np.eye(M, dtype=np.float32), DH, axis=0)
_SEGREP = np.kron(np.eye(M, dtype=np.float32), np.ones((M, M), np.float32))
_SEL = np.tile(np.eye(M, dtype=np.float32), (M, 1))


def _branch_body(x_ref, att_ref, blk_ref, segrep_ref, sel_ref,
                 w1_ref, w2_ref, w3_ref, A_ref,
                 vrow_ref, vw_ref, b1_ref, b2_ref, b3_ref, vb_ref, beta_ref,
                 o_ref):
    z16s = []
    g16s = []
    zsum = None
    for m in range(M):
        xm = x_ref[m].astype(BF16) * att_ref[m].astype(BF16)
        h = jnp.dot(xm, w1_ref[m], preferred_element_type=F32)
        h = jnp.maximum(h.astype(BF16) + b1_ref[m].astype(BF16), 0.0)
        h = jnp.dot(h, w2_ref[m], preferred_element_type=F32)
        h = jnp.maximum(h.astype(BF16) + b2_ref[m].astype(BF16), 0.0)
        z = jnp.dot(h, w3_ref[m], preferred_element_type=F32) + b3_ref[m]
        zsum = z if zsum is None else zsum + z
        z16 = z.astype(BF16)
        z16s.append(z16)
        g = jnp.dot(z16, A_ref[:], preferred_element_type=F32) + vrow_ref[:]
        g16s.append(g.astype(BF16))

    ss = []
    for i in range(M):
        p = jnp.concatenate([g16s[i] * z16s[j] for j in range(M)], axis=1)
        ss.append(jnp.dot(p, blk_ref[:], preferred_element_type=F32))
    s = jnp.concatenate(ss, axis=1)
    mx = jnp.max(s, axis=1, keepdims=True)
    e = jnp.exp(s - mx)
    d = jnp.dot(e, segrep_ref[:], preferred_element_type=F32)
    attn = e * (1.0 / d)
    wsum = jnp.dot(attn, sel_ref[:], preferred_element_type=F32)

    vin = None
    for j in range(M):
        c = wsum[:, j:j + 1].astype(BF16) * z16s[j]
        vin = c if vin is None else vin + c

    beta = beta_ref[0, 0]
    vout = jnp.dot(vin, vw_ref[:], preferred_element_type=F32) * (1.0 / M)
    o_ref[:] = zsum * (1.0 / M) + beta * (vout + vb_ref[:])


def _kernel_body(xd_ref, attd_ref, xp_ref, attp_ref, blk_ref, segrep_ref,
                 sel_ref,
                 dw1_ref, dw2_ref, dw3_ref, dqw_ref, dkwt_ref, dvw_ref,
                 db1_ref, db2_ref, db3_ref, dqb_ref, dvb_ref,
                 pw1_ref, pw2_ref, pw3_ref, pqw_ref, pkwt_ref, pvw_ref,
                 pb1_ref, pb2_ref, pb3_ref, pqb_ref, pvb_ref,
                 dbeta_ref, pbeta_ref,
                 o_ref,
                 dw1s, dw2s, dw3s, dAs, dvrows, dvws,
                 pw1s, pw2s, pw3s, pAs, pvrows, pvws):
    i = pl.program_id(0)

    @pl.when(i == 0)
    def _():
        dw1s[:] = dw1_ref[:].astype(BF16)
        dw2s[:] = dw2_ref[:].astype(BF16)
        dw3s[:] = dw3_ref[:].astype(BF16)
        dkwt16 = dkwt_ref[:].astype(BF16)
        dAs[:] = (jnp.dot(dqw_ref[:].astype(BF16), dkwt16,
                          preferred_element_type=F32) * SCALE).astype(BF16)
        dvrows[:] = jnp.dot(dqb_ref[:].astype(BF16), dkwt16,
                            preferred_element_type=F32) * SCALE
        dvws[:] = dvw_ref[:].astype(BF16)
        pw1s[:] = pw1_ref[:].astype(BF16)
        pw2s[:] = pw2_ref[:].astype(BF16)
        pw3s[:] = pw3_ref[:].astype(BF16)
        pkwt16 = pkwt_ref[:].astype(BF16)
        pAs[:] = (jnp.dot(pqw_ref[:].astype(BF16), pkwt16,
                          preferred_element_type=F32) * SCALE).astype(BF16)
        pvrows[:] = jnp.dot(pqb_ref[:].astype(BF16), pkwt16,
                            preferred_element_type=F32) * SCALE
        pvws[:] = pvw_ref[:].astype(BF16)

    @pl.when(i < DBLK)
    def _():
        _branch_body(xd_ref, attd_ref, blk_ref, segrep_ref, sel_ref,
                     dw1s, dw2s, dw3s, dAs,
                     dvrows, dvws, db1_ref, db2_ref, db3_ref, dvb_ref,
                     dbeta_ref, o_ref)

    @pl.when(i >= DBLK)
    def _():
        _branch_body(xp_ref, attp_ref, blk_ref, segrep_ref, sel_ref,
                     pw1s, pw2s, pw3s, pAs,
                     pvrows, pvws, pb1_ref, pb2_ref, pb3_ref, pvb_ref,
                     pbeta_ref, o_ref)


_VMEM = pl.BlockSpec(memory_space=pltpu.VMEM)
_SMEM = pl.BlockSpec(memory_space=pltpu.SMEM)


@jax.jit
def kernel(drug_fea_tensor, protein_fea_tensor, drug_att, protein_att, dW1,
           db1, dW2, db2, dW3, db3, pW1, pb1, pW2, pb2, pW3, pb3, QdW, Qdb,
           KdW, Kdb, VdW, Vdb, QpW, Qpb, KpW, Kpb, VpW, Vpb, beta_drug,
           beta_protein):
    out = pl.pallas_call(
        _kernel_body,
        grid=(DBLK + PBLK,),
        in_specs=(
            [pl.BlockSpec((M, BN, DIN), lambda i: (0, jnp.minimum(i, DBLK - 1), 0)),
             pl.BlockSpec((M, BN, 1), lambda i: (0, jnp.minimum(i, DBLK - 1), 0)),
             pl.BlockSpec((M, BN, DIN), lambda i: (0, jnp.maximum(i - DBLK, 0), 0)),
             pl.BlockSpec((M, BN, 1), lambda i: (0, jnp.maximum(i - DBLK, 0), 0))]
            + [_VMEM] * 25 + [_SMEM] * 2
        ),
        out_specs=pl.BlockSpec((BN, DH), lambda i: (i, 0)),
        out_shape=jax.ShapeDtypeStruct((NROWS, DH), F32),
        scratch_shapes=[
            pltpu.VMEM((M, DIN, DMID), BF16), pltpu.VMEM((M, DMID, DMID), BF16),
            pltpu.VMEM((M, DMID, DH), BF16), pltpu.VMEM((DH, DH), BF16),
            pltpu.VMEM((1, DH), F32), pltpu.VMEM((DH, DH), BF16),
            pltpu.VMEM((M, DIN, DMID), BF16), pltpu.VMEM((M, DMID, DMID), BF16),
            pltpu.VMEM((M, DMID, DH), BF16), pltpu.VMEM((DH, DH), BF16),
            pltpu.VMEM((1, DH), F32), pltpu.VMEM((DH, DH), BF16),
        ],
    )(drug_fea_tensor, drug_att, protein_fea_tensor, protein_att,
      jnp.asarray(_BLK, BF16), jnp.asarray(_SEGREP), jnp.asarray(_SEL),
      dW1, dW2, dW3, QdW, KdW.T, VdW,
      db1, db2, db3, jnp.reshape(Qdb, (1, DH)), jnp.reshape(Vdb, (1, DH)),
      pW1, pW2, pW3, QpW, KpW.T, VpW,
      pb1, pb2, pb3, jnp.reshape(Qpb, (1, DH)), jnp.reshape(Vpb, (1, DH)),
      jnp.reshape(beta_drug, (1, 1)), jnp.reshape(beta_protein, (1, 1)))

    drug_emb = out[:ND]
    protein_emb = out[DBLK * BN:DBLK * BN + NP]
    return (drug_emb, protein_emb)
